# baseline (speedup 1.0000x reference)
import functools
import jax
import jax.numpy as jnp
from jax import lax
from jax.experimental import pallas as pl
from jax.experimental.pallas import tpu as pltpu
from jax.experimental.pallas import tpu_sc as plsc

VOCAB = 1000000
DIM = 64
BATCH = 4096
SEQ = 200
B = BATCH * SEQ

NC = 2
NS = 16
NW = NC * NS
B_PER_W = B // NW

IDXROW = 128
K = 4
CHUNK = K * IDXROW
N_MACRO = B_PER_W // CHUNK
ROWS_PER_W = B_PER_W // IDXROW


def _build_gather():
    mesh = plsc.VectorSubcoreMesh(core_axis_name="c", subcore_axis_name="s")

    @functools.partial(
        pl.kernel,
        mesh=mesh,
        out_type=jax.ShapeDtypeStruct((B, DIM), jnp.float32),
        scratch_types=[
            pltpu.VMEM((K, IDXROW), jnp.int32),
            pltpu.VMEM((CHUNK, DIM), jnp.float32),
            pltpu.SemaphoreType.DMA,
        ],
        compiler_params=pltpu.CompilerParams(use_tc_tiling_on_sc=False),
    )
    def gather_kernel(ids_hbm, table_hbm, out_hbm, idx_v, rows_v, sem):
        wid = lax.axis_index("s") * NC + lax.axis_index("c")
        row_base = wid * ROWS_PER_W
        out_base = wid * B_PER_W

        def macro(m, carry):
            r0 = row_base + m * K
            o0 = out_base + m * CHUNK
            pltpu.sync_copy(ids_hbm.at[pl.ds(r0, K)], idx_v)
            copies = []
            for j in range(K):
                copies.append(
                    pltpu.async_copy(
                        table_hbm.at[idx_v.at[j]],
                        rows_v.at[pl.ds(j * IDXROW, IDXROW)],
                        sem,
                    )
                )
            for cp in copies:
                cp.wait()
            pltpu.sync_copy(rows_v, out_hbm.at[pl.ds(o0, CHUNK)])
            return carry

        lax.fori_loop(0, N_MACRO, macro, 0)

    return gather_kernel


_gather = _build_gather()


@jax.jit
def kernel(input_ids, table):
    ids2d = input_ids.astype(jnp.int32).reshape(B // IDXROW, IDXROW)
    out = _gather(ids2d, table)
    return out.reshape(BATCH, SEQ, DIM)


# speedup vs baseline: 1.0428x; 1.0428x over previous
import functools
import jax
import jax.numpy as jnp
from jax import lax
from jax.experimental import pallas as pl
from jax.experimental.pallas import tpu as pltpu
from jax.experimental.pallas import tpu_sc as plsc

VOCAB = 1000000
DIM = 64
BATCH = 4096
SEQ = 200
B = BATCH * SEQ

NC = 2
NS = 16
NW = NC * NS
B_PER_W = B // NW

IDXROW = 128
K = 4
CHUNK = K * IDXROW
N_MACRO = B_PER_W // CHUNK
ROWS_PER_W = B_PER_W // IDXROW


def _build_gather():
    mesh = plsc.VectorSubcoreMesh(core_axis_name="c", subcore_axis_name="s")

    @functools.partial(
        pl.kernel,
        mesh=mesh,
        out_type=jax.ShapeDtypeStruct((B, DIM), jnp.float32),
        scratch_types=[
            pltpu.VMEM((ROWS_PER_W, IDXROW), jnp.int32),
            pltpu.VMEM((CHUNK, DIM), jnp.float32),
            pltpu.VMEM((CHUNK, DIM), jnp.float32),
            pltpu.SemaphoreType.DMA,
            pltpu.SemaphoreType.DMA,
            pltpu.SemaphoreType.DMA,
            pltpu.SemaphoreType.DMA,
        ],
        compiler_params=pltpu.CompilerParams(use_tc_tiling_on_sc=False),
    )
    def gather_kernel(ids_hbm, table_hbm, out_hbm,
                      idx_all, rows0, rows1, g0, g1, o0, o1):
        wid = lax.axis_index("s") * NC + lax.axis_index("c")
        out_base = wid * B_PER_W
        rows = (rows0, rows1)
        gsem = (g0, g1)
        osem = (o0, o1)

        def fire_gather(m, b):
            r0 = m * K
            for j in range(K):
                pltpu.async_copy(
                    table_hbm.at[idx_all.at[r0 + j]],
                    rows[b].at[pl.ds(j * IDXROW, IDXROW)],
                    gsem[b],
                )

        def wait_gather(m, b):
            r0 = m * K
            for j in range(K):
                pltpu.make_async_copy(
                    table_hbm.at[idx_all.at[r0 + j]],
                    rows[b].at[pl.ds(j * IDXROW, IDXROW)],
                    gsem[b],
                ).wait()

        def fire_out(m, b):
            pltpu.async_copy(
                rows[b], out_hbm.at[pl.ds(out_base + m * CHUNK, CHUNK)], osem[b]
            )

        def wait_out(m, b):
            pltpu.make_async_copy(
                rows[b], out_hbm.at[pl.ds(out_base + m * CHUNK, CHUNK)], osem[b]
            ).wait()

        def step(m, b):
            nb = 1 - b
            wait_out(m - 1, nb)
            fire_gather(m + 1, nb)
            wait_gather(m, b)
            fire_out(m, b)

        pltpu.sync_copy(ids_hbm.at[pl.ds(wid * ROWS_PER_W, ROWS_PER_W)], idx_all)

        fire_gather(0, 0)
        fire_gather(1, 1)
        wait_gather(0, 0)
        fire_out(0, 0)

        def loop_body(i, carry):
            m = 1 + 2 * i
            step(m, 1)
            step(m + 1, 0)
            return carry

        lax.fori_loop(0, (N_MACRO - 2) // 2, loop_body, 0)

        wait_out(N_MACRO - 2, 0)
        wait_gather(N_MACRO - 1, 1)
        fire_out(N_MACRO - 1, 1)
        wait_out(N_MACRO - 1, 1)

    return gather_kernel


_gather = _build_gather()


@jax.jit
def kernel(input_ids, table):
    ids2d = input_ids.astype(jnp.int32).reshape(B // IDXROW, IDXROW)
    out = _gather(ids2d, table)
    return out.reshape(BATCH, SEQ, DIM)
